# baseline (speedup 1.0000x reference)
import jax
import jax.numpy as jnp
from jax.experimental import pallas as pl
from jax.experimental.pallas import tpu as pltpu

_SIZES = [256, 256, 128, 256, 256]
_NL = 4
_B = 1024
_BB = 128


def _fwd_body(*refs):
    ins = refs[:4 * _NL]
    x_ref = refs[4 * _NL]
    out_ref = refs[4 * _NL + 1]
    pq = refs[4 * _NL + 2:]

    @pl.when(pl.program_id(0) == 0)
    def _prep():
        for l in range(_NL):
            a0 = ins[4 * l][...]
            a1 = ins[4 * l + 1][...]
            m = a1 > a0
            f = ins[4 * l + 3][...] == 0
            sign = jnp.where(f, -1.0, 1.0)
            pv = jnp.where(m, sign, 0.0)
            qv = jnp.where(m & f, 1.0, 0.0)
            pq[2 * l][...] = pv.T.astype(jnp.bfloat16)
            pq[2 * l + 1][...] = qv.T.astype(jnp.bfloat16)

    x = x_ref[...].astype(jnp.bfloat16)
    for l in range(_NL):
        fin = _SIZES[l]
        fout = _SIZES[l + 1]
        PT = pq[2 * l][...]
        QT = pq[2 * l + 1][...]
        acc = jnp.zeros((_BB, fout), dtype=jnp.bfloat16)
        for i in range(fin):
            acc = jnp.maximum(acc, x[:, i:i + 1] * PT[i:i + 1, :] + QT[i:i + 1, :])
        f = ins[4 * l + 2][...] == 0
        x = jnp.where(f, jnp.bfloat16(1.0) - acc, acc)
    out_ref[...] = x.astype(jnp.float32)


def kernel(x, logits_0, logits_1, logits_2, logits_3,
           ops_0, ops_1, ops_2, ops_3,
           gnoise_0, gnoise_1, gnoise_2, gnoise_3):
    logits = [logits_0, logits_1, logits_2, logits_3]
    gnoise = [gnoise_0, gnoise_1, gnoise_2, gnoise_3]
    ops = [ops_0, ops_1, ops_2, ops_3]

    fwd_in = []
    fwd_specs = []
    for l in range(_NL):
        a = logits[l] + gnoise[l]
        fwd_in.append(a[:, :, 0])
        fwd_in.append(a[:, :, 1])
        fwd_in.append(ops[l].reshape(1, -1))
        fwd_in.append(ops[l].reshape(-1, 1))
        fin, fout = _SIZES[l], _SIZES[l + 1]
        fwd_specs += [
            pl.BlockSpec((fout, fin), lambda j: (0, 0)),
            pl.BlockSpec((fout, fin), lambda j: (0, 0)),
            pl.BlockSpec((1, fout), lambda j: (0, 0)),
            pl.BlockSpec((fout, 1), lambda j: (0, 0)),
        ]
    fwd_in.append(x)
    fwd_specs.append(pl.BlockSpec((_BB, _SIZES[0]), lambda j: (j, 0)))

    scratch = []
    for l in range(_NL):
        fin, fout = _SIZES[l], _SIZES[l + 1]
        scratch += [pltpu.VMEM((fin, fout), jnp.bfloat16)] * 2

    y = pl.pallas_call(
        _fwd_body,
        grid=(_B // _BB,),
        in_specs=fwd_specs,
        out_specs=pl.BlockSpec((_BB, _SIZES[_NL]), lambda j: (j, 0)),
        out_shape=jax.ShapeDtypeStruct((_B, _SIZES[_NL]), jnp.float32),
        scratch_shapes=scratch,
        compiler_params=pltpu.CompilerParams(
            dimension_semantics=("arbitrary",),
        ),
    )(*fwd_in)

    return y
